# baseline (speedup 1.0000x reference)
import jax
import jax.numpy as jnp
from jax import lax
from jax.experimental import pallas as pl
from jax.experimental.pallas import tpu as pltpu
from jax.experimental.pallas import tpu_sc as plsc

N = 10000
E = 320000
D = 128
NC = 2
NS = 16
NW = NC * NS
CHUNK = 128
TOTCH = 2560
PCHUNK = 8
K0 = 152
K1 = 8
E_PAD = TOTCH * CHUNK
N_PAD = 10240
DPT = N_PAD // NS
NA = 10240
RPT = NA // NS
DR = 128

_sc_mesh = plsc.VectorSubcoreMesh(core_axis_name="c", subcore_axis_name="s")


def _deg_body(dstp, wp, z1d, deg_out, dst_v, w_v, dbuf, deg_sh):
    cid = lax.axis_index("c")
    sid = lax.axis_index("s")
    wid = sid * NC + cid
    base_ch = wid * (TOTCH // NW)
    pltpu.sync_copy(z1d, deg_sh.at[pl.ds(sid * DPT, DPT)])
    plsc.subcore_barrier()

    def do_pass(p, carry):
        pltpu.sync_copy(dstp.at[pl.ds(base_ch + p * PCHUNK, PCHUNK)], dst_v)
        pltpu.sync_copy(wp.at[pl.ds(base_ch + p * PCHUNK, PCHUNK)], w_v)

        def chunk(c, carry2):
            pltpu.sync_copy(w_v.at[c], deg_sh.at[dst_v.at[c]], add=True)
            return carry2

        lax.fori_loop(0, PCHUNK, chunk, 0)
        return carry

    lax.fori_loop(0, (TOTCH // NW) // PCHUNK, do_pass, 0)
    plsc.subcore_barrier()
    sl = pl.ds(sid * DPT, DPT)
    pltpu.sync_copy(deg_sh.at[sl], dbuf)
    pltpu.sync_copy(dbuf, deg_out.at[cid, sl])


_deg_kernel = pl.kernel(
    _deg_body,
    out_type=jax.ShapeDtypeStruct((NC, N_PAD), jnp.float32),
    mesh=_sc_mesh,
    scratch_types=[
        pltpu.VMEM((PCHUNK, CHUNK), jnp.int32),
        pltpu.VMEM((PCHUNK, CHUNK), jnp.float32),
        pltpu.VMEM((DPT,), jnp.float32),
        pltpu.VMEM_SHARED((N_PAD,), jnp.float32),
    ],
)


def _agg_body(g, srcp, dstp, wp, agg_out,
              src_v, dst_v, w_v, rows0, rows1, agg_sh, sem0, sem1):
    cid = lax.axis_index("c")
    sid = lax.axis_index("s")

    with jax.named_scope("agg_zero"):
        def zrow(r, carry):
            for j in range(8):
                rows0[r, pl.ds(j * 16, 16)] = jnp.zeros((16,), jnp.float32)
            return carry

        lax.fori_loop(0, DR, zrow, 0)
        for k in range(RPT // DR):
            pltpu.sync_copy(rows0, agg_sh.at[pl.ds(sid * RPT + k * DR, DR)])
        plsc.subcore_barrier()

    def scale(rows, c):
        def grp(b, carry):
            wv = w_v[c, pl.ds(b * 16, 16)]
            for ei in range(16):
                e = b * 16 + ei
                wsp = jnp.broadcast_to(wv[ei], (16,))
                for j in range(8):
                    sl = pl.ds(j * 16, 16)
                    rows[e, sl] = rows[e, sl] * wsp
            return carry
        lax.fori_loop(0, 8, grp, 0)

    def run(base_ch, npass):
        def do_pass(p, carry):
            st = base_ch + p * PCHUNK
            pltpu.sync_copy(srcp.at[pl.ds(st, PCHUNK)], src_v)
            pltpu.sync_copy(dstp.at[pl.ds(st, PCHUNK)], dst_v)
            pltpu.sync_copy(wp.at[pl.ds(st, PCHUNK)], w_v)
            pltpu.async_copy(g.at[src_v.at[0]], rows0, sem0)

            def body(i, carry2):
                c0 = 2 * i
                c1 = c0 + 1
                pltpu.make_async_copy(g.at[src_v.at[c0]], rows0, sem0).wait()
                pltpu.async_copy(g.at[src_v.at[c1]], rows1, sem1)
                scale(rows0, c0)
                pltpu.sync_copy(rows0, agg_sh.at[dst_v.at[c0]], add=True)
                pltpu.make_async_copy(g.at[src_v.at[c1]], rows1, sem1).wait()

                @pl.when(i < PCHUNK // 2 - 1)
                def _():
                    pltpu.async_copy(g.at[src_v.at[c0 + 2]], rows0, sem0)

                scale(rows1, c1)
                pltpu.sync_copy(rows1, agg_sh.at[dst_v.at[c1]], add=True)
                return carry2

            lax.fori_loop(0, PCHUNK // 2, body, 0)
            return carry

        lax.fori_loop(0, npass, do_pass, 0)

    with jax.named_scope("agg_edges"):
        @pl.when(cid == 0)
        def _():
            run(sid * K0, K0 // PCHUNK)

        @pl.when(cid == 1)
        def _():
            run(NS * K0 + sid * K1, K1 // PCHUNK)

        plsc.subcore_barrier()

    with jax.named_scope("agg_drain"):
        for k in range(RPT // DR):
            sl = pl.ds(sid * RPT + k * DR, DR)
            bsl = pl.ds(0, DR)
            pltpu.sync_copy(agg_sh.at[sl], rows0.at[bsl])
            pltpu.sync_copy(rows0.at[bsl], agg_out.at[cid, sl])


_agg_kernel = pl.kernel(
    _agg_body,
    out_type=jax.ShapeDtypeStruct((NC, NA, D), jnp.float32),
    mesh=_sc_mesh,
    scratch_types=[
        pltpu.VMEM((PCHUNK, CHUNK), jnp.int32),
        pltpu.VMEM((PCHUNK, CHUNK), jnp.int32),
        pltpu.VMEM((PCHUNK, CHUNK), jnp.float32),
        pltpu.VMEM((CHUNK, D), jnp.float32),
        pltpu.VMEM((CHUNK, D), jnp.float32),
        pltpu.VMEM_SHARED((NA, D), jnp.float32),
        pltpu.SemaphoreType.DMA,
        pltpu.SemaphoreType.DMA,
    ],
)


BLK = 2000


def _prep_body(x_ref, wi_ref, wr_ref, b_ref, dA_ref, dB_ref,
               g_ref, xr_ref, dis_ref):
    deg = dA_ref[...] + dB_ref[...]
    dis = jnp.where(deg > 0, lax.rsqrt(jnp.maximum(deg, 1e-30)), 0.0)
    xb = x_ref[...]
    g_ref[...] = dis * jnp.dot(xb, wi_ref[...], preferred_element_type=jnp.float32)
    xr_ref[...] = jnp.dot(xb, wr_ref[...], preferred_element_type=jnp.float32) + b_ref[...]
    dis_ref[...] = dis


_prep = pl.pallas_call(
    _prep_body,
    grid=(N // BLK,),
    in_specs=[
        pl.BlockSpec((BLK, D), lambda i: (i, 0)),
        pl.BlockSpec((D, D), lambda i: (0, 0)),
        pl.BlockSpec((D, D), lambda i: (0, 0)),
        pl.BlockSpec((1, D), lambda i: (0, 0)),
        pl.BlockSpec((BLK, 1), lambda i: (i, 0)),
        pl.BlockSpec((BLK, 1), lambda i: (i, 0)),
    ],
    out_specs=[
        pl.BlockSpec((BLK, D), lambda i: (i, 0)),
        pl.BlockSpec((BLK, D), lambda i: (i, 0)),
        pl.BlockSpec((BLK, 1), lambda i: (i, 0)),
    ],
    out_shape=[
        jax.ShapeDtypeStruct((N, D), jnp.float32),
        jax.ShapeDtypeStruct((N, D), jnp.float32),
        jax.ShapeDtypeStruct((N, 1), jnp.float32),
    ],
)


def _mid_body(aA_ref, aB_ref, xr_ref, dis_ref, wi_ref, wr_ref, b_ref,
              g_ref, xr2_ref):
    dis = dis_ref[...]
    out = jnp.maximum(dis * (aA_ref[...] + aB_ref[...]) + xr_ref[...], 0.0)
    g_ref[...] = dis * jnp.dot(out, wi_ref[...], preferred_element_type=jnp.float32)
    xr2_ref[...] = jnp.dot(out, wr_ref[...], preferred_element_type=jnp.float32) + b_ref[...]


_mid = pl.pallas_call(
    _mid_body,
    grid=(N // BLK,),
    in_specs=[
        pl.BlockSpec((BLK, D), lambda i: (i, 0)),
        pl.BlockSpec((BLK, D), lambda i: (i, 0)),
        pl.BlockSpec((BLK, D), lambda i: (i, 0)),
        pl.BlockSpec((BLK, 1), lambda i: (i, 0)),
        pl.BlockSpec((D, D), lambda i: (0, 0)),
        pl.BlockSpec((D, D), lambda i: (0, 0)),
        pl.BlockSpec((1, D), lambda i: (0, 0)),
    ],
    out_specs=[
        pl.BlockSpec((BLK, D), lambda i: (i, 0)),
        pl.BlockSpec((BLK, D), lambda i: (i, 0)),
    ],
    out_shape=[
        jax.ShapeDtypeStruct((N, D), jnp.float32),
        jax.ShapeDtypeStruct((N, D), jnp.float32),
    ],
)


def _fin_body(aA_ref, aB_ref, xr_ref, dis_ref, o_ref):
    o_ref[...] = jnp.maximum(
        dis_ref[...] * (aA_ref[...] + aB_ref[...]) + xr_ref[...], 0.0)


_fin = pl.pallas_call(
    _fin_body,
    grid=(N // BLK,),
    in_specs=[
        pl.BlockSpec((BLK, D), lambda i: (i, 0)),
        pl.BlockSpec((BLK, D), lambda i: (i, 0)),
        pl.BlockSpec((BLK, D), lambda i: (i, 0)),
        pl.BlockSpec((BLK, 1), lambda i: (i, 0)),
    ],
    out_specs=pl.BlockSpec((BLK, D), lambda i: (i, 0)),
    out_shape=jax.ShapeDtypeStruct((N, D), jnp.float32),
)


def kernel(x, edge_index, edge_attr, Wi1, Wr1, b1, Wi2, Wr2, b2, Wi3, Wr3, b3):
    pad = E_PAD - E
    src = jnp.concatenate([edge_index[0], jnp.zeros((pad,), jnp.int32)])
    dst = jnp.concatenate([edge_index[1], jnp.zeros((pad,), jnp.int32)])
    w = jnp.concatenate([edge_attr, jnp.zeros((pad,), jnp.float32)])
    srcp = src.reshape(TOTCH, CHUNK)
    dstp = dst.reshape(TOTCH, CHUNK)
    wp = w.reshape(TOTCH, CHUNK)
    z1d = jnp.zeros((DPT,), jnp.float32)

    deg2 = _deg_kernel(dstp, wp, z1d)
    degA = deg2[0, :N].reshape(N, 1)
    degB = deg2[1, :N].reshape(N, 1)

    g1, xr1, dis = _prep(x, Wi1, Wr1, b1.reshape(1, D), degA, degB)
    agg1 = _agg_kernel(g1, srcp, dstp, wp)
    g2, xr2 = _mid(agg1[0], agg1[1], xr1, dis, Wi2, Wr2, b2.reshape(1, D))
    agg2 = _agg_kernel(g2, srcp, dstp, wp)
    g3, xr3 = _mid(agg2[0], agg2[1], xr2, dis, Wi3, Wr3, b3.reshape(1, D))
    agg3 = _agg_kernel(g3, srcp, dstp, wp)
    return _fin(agg3[0], agg3[1], xr3, dis)


# speedup vs baseline: 1.0762x; 1.0762x over previous
import jax
import jax.numpy as jnp
from jax import lax
from jax.experimental import pallas as pl
from jax.experimental.pallas import tpu as pltpu
from jax.experimental.pallas import tpu_sc as plsc

N = 10000
E = 320000
D = 128
NC = 2
NS = 16
NW = NC * NS
CHUNK = 128
TOTCH = 2560
PCHUNK = 8
K0 = 144
K1 = 16
E_PAD = TOTCH * CHUNK
N_PAD = 10240
DPT = N_PAD // NS
NA = 10240
RPT = NA // NS
DR = 128

_sc_mesh = plsc.VectorSubcoreMesh(core_axis_name="c", subcore_axis_name="s")


def _deg_body(dstp, wp, z1d, deg_out, dst_v, w_v, dbuf, deg_sh):
    cid = lax.axis_index("c")
    sid = lax.axis_index("s")
    wid = sid * NC + cid
    base_ch = wid * (TOTCH // NW)
    pltpu.sync_copy(z1d, deg_sh.at[pl.ds(sid * DPT, DPT)])
    plsc.subcore_barrier()

    def do_pass(p, carry):
        pltpu.sync_copy(dstp.at[pl.ds(base_ch + p * PCHUNK, PCHUNK)], dst_v)
        pltpu.sync_copy(wp.at[pl.ds(base_ch + p * PCHUNK, PCHUNK)], w_v)

        def chunk(c, carry2):
            pltpu.sync_copy(w_v.at[c], deg_sh.at[dst_v.at[c]], add=True)
            return carry2

        lax.fori_loop(0, PCHUNK, chunk, 0)
        return carry

    lax.fori_loop(0, (TOTCH // NW) // PCHUNK, do_pass, 0)
    plsc.subcore_barrier()
    sl = pl.ds(sid * DPT, DPT)
    pltpu.sync_copy(deg_sh.at[sl], dbuf)
    pltpu.sync_copy(dbuf, deg_out.at[cid, sl])


_deg_kernel = pl.kernel(
    _deg_body,
    out_type=jax.ShapeDtypeStruct((NC, N_PAD), jnp.float32),
    mesh=_sc_mesh,
    scratch_types=[
        pltpu.VMEM((PCHUNK, CHUNK), jnp.int32),
        pltpu.VMEM((PCHUNK, CHUNK), jnp.float32),
        pltpu.VMEM((DPT,), jnp.float32),
        pltpu.VMEM_SHARED((N_PAD,), jnp.float32),
    ],
)


def _agg_body(g, srcp, dstp, wp, agg_out,
              src_v, dst_v, w_v, rows0, rows1, agg_sh, sem0, sem1, ssem):
    cid = lax.axis_index("c")
    sid = lax.axis_index("s")

    with jax.named_scope("agg_zero"):
        def zrow(r, carry):
            for j in range(8):
                rows0[r, pl.ds(j * 16, 16)] = jnp.zeros((16,), jnp.float32)
            return carry

        lax.fori_loop(0, DR, zrow, 0)
        for k in range(RPT // DR):
            pltpu.sync_copy(rows0, agg_sh.at[pl.ds(sid * RPT + k * DR, DR)])
        plsc.subcore_barrier()

    def scale(rows, c):
        def grp(b, carry):
            wv = w_v[c, pl.ds(b * 16, 16)]
            for ei in range(16):
                e = b * 16 + ei
                wsp = jnp.broadcast_to(wv[ei], (16,))
                for j in range(8):
                    sl = pl.ds(j * 16, 16)
                    rows[e, sl] = rows[e, sl] * wsp
            return carry
        lax.fori_loop(0, 8, grp, 0)

    def run(base_ch, npass):
        def do_pass(p, carry):
            st = base_ch + p * PCHUNK
            pltpu.sync_copy(srcp.at[pl.ds(st, PCHUNK)], src_v)
            pltpu.sync_copy(dstp.at[pl.ds(st, PCHUNK)], dst_v)
            pltpu.sync_copy(wp.at[pl.ds(st, PCHUNK)], w_v)
            pltpu.async_copy(g.at[src_v.at[0]], rows0, sem0)

            def body(i, carry2):
                c0 = 2 * i
                c1 = c0 + 1
                pltpu.make_async_copy(g.at[src_v.at[c0]], rows0, sem0).wait()
                pltpu.async_copy(g.at[src_v.at[c1]], rows1, sem1)
                scale(rows0, c0)
                pltpu.async_copy(rows0, agg_sh.at[dst_v.at[c0]], ssem, add=True)
                pltpu.make_async_copy(g.at[src_v.at[c1]], rows1, sem1).wait()
                scale(rows1, c1)
                pltpu.make_async_copy(rows0, agg_sh.at[dst_v.at[c0]], ssem).wait()

                @pl.when(i < PCHUNK // 2 - 1)
                def _():
                    pltpu.async_copy(g.at[src_v.at[c0 + 2]], rows0, sem0)

                pltpu.sync_copy(rows1, agg_sh.at[dst_v.at[c1]], add=True)
                return carry2

            lax.fori_loop(0, PCHUNK // 2, body, 0)
            return carry

        lax.fori_loop(0, npass, do_pass, 0)

    with jax.named_scope("agg_edges"):
        @pl.when(cid == 0)
        def _():
            run(sid * K0, K0 // PCHUNK)

        @pl.when(cid == 1)
        def _():
            run(NS * K0 + sid * K1, K1 // PCHUNK)

        plsc.subcore_barrier()

    with jax.named_scope("agg_drain"):
        for k in range(RPT // DR):
            sl = pl.ds(sid * RPT + k * DR, DR)
            bsl = pl.ds(0, DR)
            pltpu.sync_copy(agg_sh.at[sl], rows0.at[bsl])
            pltpu.sync_copy(rows0.at[bsl], agg_out.at[cid, sl])


_agg_kernel = pl.kernel(
    _agg_body,
    out_type=jax.ShapeDtypeStruct((NC, NA, D), jnp.float32),
    mesh=_sc_mesh,
    scratch_types=[
        pltpu.VMEM((PCHUNK, CHUNK), jnp.int32),
        pltpu.VMEM((PCHUNK, CHUNK), jnp.int32),
        pltpu.VMEM((PCHUNK, CHUNK), jnp.float32),
        pltpu.VMEM((CHUNK, D), jnp.float32),
        pltpu.VMEM((CHUNK, D), jnp.float32),
        pltpu.VMEM_SHARED((NA, D), jnp.float32),
        pltpu.SemaphoreType.DMA,
        pltpu.SemaphoreType.DMA,
        pltpu.SemaphoreType.DMA,
    ],
)


BLK = 2000


def _prep_body(x_ref, wi_ref, wr_ref, b_ref, dA_ref, dB_ref,
               g_ref, xr_ref, dis_ref):
    deg = dA_ref[...] + dB_ref[...]
    dis = jnp.where(deg > 0, lax.rsqrt(jnp.maximum(deg, 1e-30)), 0.0)
    xb = x_ref[...]
    g_ref[...] = dis * jnp.dot(xb, wi_ref[...], preferred_element_type=jnp.float32)
    xr_ref[...] = jnp.dot(xb, wr_ref[...], preferred_element_type=jnp.float32) + b_ref[...]
    dis_ref[...] = dis


_prep = pl.pallas_call(
    _prep_body,
    grid=(N // BLK,),
    in_specs=[
        pl.BlockSpec((BLK, D), lambda i: (i, 0)),
        pl.BlockSpec((D, D), lambda i: (0, 0)),
        pl.BlockSpec((D, D), lambda i: (0, 0)),
        pl.BlockSpec((1, D), lambda i: (0, 0)),
        pl.BlockSpec((BLK, 1), lambda i: (i, 0)),
        pl.BlockSpec((BLK, 1), lambda i: (i, 0)),
    ],
    out_specs=[
        pl.BlockSpec((BLK, D), lambda i: (i, 0)),
        pl.BlockSpec((BLK, D), lambda i: (i, 0)),
        pl.BlockSpec((BLK, 1), lambda i: (i, 0)),
    ],
    out_shape=[
        jax.ShapeDtypeStruct((N, D), jnp.float32),
        jax.ShapeDtypeStruct((N, D), jnp.float32),
        jax.ShapeDtypeStruct((N, 1), jnp.float32),
    ],
)


def _mid_body(aA_ref, aB_ref, xr_ref, dis_ref, wi_ref, wr_ref, b_ref,
              g_ref, xr2_ref):
    dis = dis_ref[...]
    out = jnp.maximum(dis * (aA_ref[...] + aB_ref[...]) + xr_ref[...], 0.0)
    g_ref[...] = dis * jnp.dot(out, wi_ref[...], preferred_element_type=jnp.float32)
    xr2_ref[...] = jnp.dot(out, wr_ref[...], preferred_element_type=jnp.float32) + b_ref[...]


_mid = pl.pallas_call(
    _mid_body,
    grid=(N // BLK,),
    in_specs=[
        pl.BlockSpec((BLK, D), lambda i: (i, 0)),
        pl.BlockSpec((BLK, D), lambda i: (i, 0)),
        pl.BlockSpec((BLK, D), lambda i: (i, 0)),
        pl.BlockSpec((BLK, 1), lambda i: (i, 0)),
        pl.BlockSpec((D, D), lambda i: (0, 0)),
        pl.BlockSpec((D, D), lambda i: (0, 0)),
        pl.BlockSpec((1, D), lambda i: (0, 0)),
    ],
    out_specs=[
        pl.BlockSpec((BLK, D), lambda i: (i, 0)),
        pl.BlockSpec((BLK, D), lambda i: (i, 0)),
    ],
    out_shape=[
        jax.ShapeDtypeStruct((N, D), jnp.float32),
        jax.ShapeDtypeStruct((N, D), jnp.float32),
    ],
)


def _fin_body(aA_ref, aB_ref, xr_ref, dis_ref, o_ref):
    o_ref[...] = jnp.maximum(
        dis_ref[...] * (aA_ref[...] + aB_ref[...]) + xr_ref[...], 0.0)


_fin = pl.pallas_call(
    _fin_body,
    grid=(N // BLK,),
    in_specs=[
        pl.BlockSpec((BLK, D), lambda i: (i, 0)),
        pl.BlockSpec((BLK, D), lambda i: (i, 0)),
        pl.BlockSpec((BLK, D), lambda i: (i, 0)),
        pl.BlockSpec((BLK, 1), lambda i: (i, 0)),
    ],
    out_specs=pl.BlockSpec((BLK, D), lambda i: (i, 0)),
    out_shape=jax.ShapeDtypeStruct((N, D), jnp.float32),
)


def kernel(x, edge_index, edge_attr, Wi1, Wr1, b1, Wi2, Wr2, b2, Wi3, Wr3, b3):
    pad = E_PAD - E
    src = jnp.concatenate([edge_index[0], jnp.zeros((pad,), jnp.int32)])
    dst = jnp.concatenate([edge_index[1], jnp.zeros((pad,), jnp.int32)])
    w = jnp.concatenate([edge_attr, jnp.zeros((pad,), jnp.float32)])
    srcp = src.reshape(TOTCH, CHUNK)
    dstp = dst.reshape(TOTCH, CHUNK)
    wp = w.reshape(TOTCH, CHUNK)
    z1d = jnp.zeros((DPT,), jnp.float32)

    deg2 = _deg_kernel(dstp, wp, z1d)
    degA = deg2[0, :N].reshape(N, 1)
    degB = deg2[1, :N].reshape(N, 1)

    g1, xr1, dis = _prep(x, Wi1, Wr1, b1.reshape(1, D), degA, degB)
    agg1 = _agg_kernel(g1, srcp, dstp, wp)
    g2, xr2 = _mid(agg1[0], agg1[1], xr1, dis, Wi2, Wr2, b2.reshape(1, D))
    agg2 = _agg_kernel(g2, srcp, dstp, wp)
    g3, xr3 = _mid(agg2[0], agg2[1], xr2, dis, Wi3, Wr3, b3.reshape(1, D))
    agg3 = _agg_kernel(g3, srcp, dstp, wp)
    return _fin(agg3[0], agg3[1], xr3, dis)


# speedup vs baseline: 1.0811x; 1.0046x over previous
import jax
import jax.numpy as jnp
from jax import lax
from jax.experimental import pallas as pl
from jax.experimental.pallas import tpu as pltpu
from jax.experimental.pallas import tpu_sc as plsc

N = 10000
E = 320000
D = 128
NC = 2
NS = 16
NW = NC * NS
CHUNK = 128
TOTCH = 2560
PCHUNK = 16
K0 = 144
K1 = 16
E_PAD = TOTCH * CHUNK
N_PAD = 10240
DPT = N_PAD // NS
NA = 10240
RPT = NA // NS
DR = 128

_sc_mesh = plsc.VectorSubcoreMesh(core_axis_name="c", subcore_axis_name="s")


def _deg_body(dstp, wp, z1d, deg_out, dst_v, w_v, dbuf, deg_sh):
    cid = lax.axis_index("c")
    sid = lax.axis_index("s")
    wid = sid * NC + cid
    base_ch = wid * (TOTCH // NW)
    pltpu.sync_copy(z1d, deg_sh.at[pl.ds(sid * DPT, DPT)])
    plsc.subcore_barrier()

    def do_pass(p, carry):
        pltpu.sync_copy(dstp.at[pl.ds(base_ch + p * PCHUNK, PCHUNK)], dst_v)
        pltpu.sync_copy(wp.at[pl.ds(base_ch + p * PCHUNK, PCHUNK)], w_v)

        def chunk(c, carry2):
            pltpu.sync_copy(w_v.at[c], deg_sh.at[dst_v.at[c]], add=True)
            return carry2

        lax.fori_loop(0, PCHUNK, chunk, 0)
        return carry

    lax.fori_loop(0, (TOTCH // NW) // PCHUNK, do_pass, 0)
    plsc.subcore_barrier()
    sl = pl.ds(sid * DPT, DPT)
    pltpu.sync_copy(deg_sh.at[sl], dbuf)
    pltpu.sync_copy(dbuf, deg_out.at[cid, sl])


_deg_kernel = pl.kernel(
    _deg_body,
    out_type=jax.ShapeDtypeStruct((NC, N_PAD), jnp.float32),
    mesh=_sc_mesh,
    scratch_types=[
        pltpu.VMEM((PCHUNK, CHUNK), jnp.int32),
        pltpu.VMEM((PCHUNK, CHUNK), jnp.float32),
        pltpu.VMEM((DPT,), jnp.float32),
        pltpu.VMEM_SHARED((N_PAD,), jnp.float32),
    ],
)


def _agg_body(g, srcp, dstp, wp, agg_out,
              src_v, dst_v, w_v, rows0, rows1, agg_sh, sem0, sem1, ssem):
    cid = lax.axis_index("c")
    sid = lax.axis_index("s")

    with jax.named_scope("agg_zero"):
        def zrow(r, carry):
            for j in range(8):
                rows0[r, pl.ds(j * 16, 16)] = jnp.zeros((16,), jnp.float32)
            return carry

        lax.fori_loop(0, DR, zrow, 0)
        for k in range(RPT // DR):
            pltpu.sync_copy(rows0, agg_sh.at[pl.ds(sid * RPT + k * DR, DR)])
        plsc.subcore_barrier()

    def scale(rows, c):
        def grp(b, carry):
            wv = w_v[c, pl.ds(b * 16, 16)]
            for ei in range(16):
                e = b * 16 + ei
                wsp = jnp.broadcast_to(wv[ei], (16,))
                for j in range(8):
                    sl = pl.ds(j * 16, 16)
                    rows[e, sl] = rows[e, sl] * wsp
            return carry
        lax.fori_loop(0, 8, grp, 0)

    def run(base_ch, npass):
        def do_pass(p, carry):
            st = base_ch + p * PCHUNK
            pltpu.sync_copy(srcp.at[pl.ds(st, PCHUNK)], src_v)
            pltpu.sync_copy(dstp.at[pl.ds(st, PCHUNK)], dst_v)
            pltpu.sync_copy(wp.at[pl.ds(st, PCHUNK)], w_v)
            pltpu.async_copy(g.at[src_v.at[0]], rows0, sem0)

            def body(i, carry2):
                c0 = 2 * i
                c1 = c0 + 1
                pltpu.async_copy(g.at[src_v.at[c1]], rows1, sem1)
                pltpu.make_async_copy(g.at[src_v.at[c0]], rows0, sem0).wait()
                scale(rows0, c0)
                pltpu.async_copy(rows0, agg_sh.at[dst_v.at[c0]], ssem, add=True)
                pltpu.make_async_copy(g.at[src_v.at[c1]], rows1, sem1).wait()
                scale(rows1, c1)
                pltpu.make_async_copy(rows0, agg_sh.at[dst_v.at[c0]], ssem).wait()

                @pl.when(i < PCHUNK // 2 - 1)
                def _():
                    pltpu.async_copy(g.at[src_v.at[c0 + 2]], rows0, sem0)

                pltpu.sync_copy(rows1, agg_sh.at[dst_v.at[c1]], add=True)
                return carry2

            lax.fori_loop(0, PCHUNK // 2, body, 0)
            return carry

        lax.fori_loop(0, npass, do_pass, 0)

    with jax.named_scope("agg_edges"):
        @pl.when(cid == 0)
        def _():
            run(sid * K0, K0 // PCHUNK)

        @pl.when(cid == 1)
        def _():
            run(NS * K0 + sid * K1, K1 // PCHUNK)

        plsc.subcore_barrier()

    with jax.named_scope("agg_drain"):
        for k in range(RPT // DR):
            sl = pl.ds(sid * RPT + k * DR, DR)
            bsl = pl.ds(0, DR)
            pltpu.sync_copy(agg_sh.at[sl], rows0.at[bsl])
            pltpu.sync_copy(rows0.at[bsl], agg_out.at[cid, sl])


_agg_kernel = pl.kernel(
    _agg_body,
    out_type=jax.ShapeDtypeStruct((NC, NA, D), jnp.float32),
    mesh=_sc_mesh,
    scratch_types=[
        pltpu.VMEM((PCHUNK, CHUNK), jnp.int32),
        pltpu.VMEM((PCHUNK, CHUNK), jnp.int32),
        pltpu.VMEM((PCHUNK, CHUNK), jnp.float32),
        pltpu.VMEM((CHUNK, D), jnp.float32),
        pltpu.VMEM((CHUNK, D), jnp.float32),
        pltpu.VMEM_SHARED((NA, D), jnp.float32),
        pltpu.SemaphoreType.DMA,
        pltpu.SemaphoreType.DMA,
        pltpu.SemaphoreType.DMA,
    ],
)


BLK = 2000


def _prep_body(x_ref, wi_ref, wr_ref, b_ref, dA_ref, dB_ref,
               g_ref, xr_ref, dis_ref):
    deg = dA_ref[...] + dB_ref[...]
    dis = jnp.where(deg > 0, lax.rsqrt(jnp.maximum(deg, 1e-30)), 0.0)
    xb = x_ref[...]
    g_ref[...] = dis * jnp.dot(xb, wi_ref[...], preferred_element_type=jnp.float32)
    xr_ref[...] = jnp.dot(xb, wr_ref[...], preferred_element_type=jnp.float32) + b_ref[...]
    dis_ref[...] = dis


_prep = pl.pallas_call(
    _prep_body,
    grid=(N // BLK,),
    in_specs=[
        pl.BlockSpec((BLK, D), lambda i: (i, 0)),
        pl.BlockSpec((D, D), lambda i: (0, 0)),
        pl.BlockSpec((D, D), lambda i: (0, 0)),
        pl.BlockSpec((1, D), lambda i: (0, 0)),
        pl.BlockSpec((BLK, 1), lambda i: (i, 0)),
        pl.BlockSpec((BLK, 1), lambda i: (i, 0)),
    ],
    out_specs=[
        pl.BlockSpec((BLK, D), lambda i: (i, 0)),
        pl.BlockSpec((BLK, D), lambda i: (i, 0)),
        pl.BlockSpec((BLK, 1), lambda i: (i, 0)),
    ],
    out_shape=[
        jax.ShapeDtypeStruct((N, D), jnp.float32),
        jax.ShapeDtypeStruct((N, D), jnp.float32),
        jax.ShapeDtypeStruct((N, 1), jnp.float32),
    ],
)


def _mid_body(aA_ref, aB_ref, xr_ref, dis_ref, wi_ref, wr_ref, b_ref,
              g_ref, xr2_ref):
    dis = dis_ref[...]
    out = jnp.maximum(dis * (aA_ref[...] + aB_ref[...]) + xr_ref[...], 0.0)
    g_ref[...] = dis * jnp.dot(out, wi_ref[...], preferred_element_type=jnp.float32)
    xr2_ref[...] = jnp.dot(out, wr_ref[...], preferred_element_type=jnp.float32) + b_ref[...]


_mid = pl.pallas_call(
    _mid_body,
    grid=(N // BLK,),
    in_specs=[
        pl.BlockSpec((BLK, D), lambda i: (i, 0)),
        pl.BlockSpec((BLK, D), lambda i: (i, 0)),
        pl.BlockSpec((BLK, D), lambda i: (i, 0)),
        pl.BlockSpec((BLK, 1), lambda i: (i, 0)),
        pl.BlockSpec((D, D), lambda i: (0, 0)),
        pl.BlockSpec((D, D), lambda i: (0, 0)),
        pl.BlockSpec((1, D), lambda i: (0, 0)),
    ],
    out_specs=[
        pl.BlockSpec((BLK, D), lambda i: (i, 0)),
        pl.BlockSpec((BLK, D), lambda i: (i, 0)),
    ],
    out_shape=[
        jax.ShapeDtypeStruct((N, D), jnp.float32),
        jax.ShapeDtypeStruct((N, D), jnp.float32),
    ],
)


def _fin_body(aA_ref, aB_ref, xr_ref, dis_ref, o_ref):
    o_ref[...] = jnp.maximum(
        dis_ref[...] * (aA_ref[...] + aB_ref[...]) + xr_ref[...], 0.0)


_fin = pl.pallas_call(
    _fin_body,
    grid=(N // BLK,),
    in_specs=[
        pl.BlockSpec((BLK, D), lambda i: (i, 0)),
        pl.BlockSpec((BLK, D), lambda i: (i, 0)),
        pl.BlockSpec((BLK, D), lambda i: (i, 0)),
        pl.BlockSpec((BLK, 1), lambda i: (i, 0)),
    ],
    out_specs=pl.BlockSpec((BLK, D), lambda i: (i, 0)),
    out_shape=jax.ShapeDtypeStruct((N, D), jnp.float32),
)


def kernel(x, edge_index, edge_attr, Wi1, Wr1, b1, Wi2, Wr2, b2, Wi3, Wr3, b3):
    pad = E_PAD - E
    src = jnp.concatenate([edge_index[0], jnp.zeros((pad,), jnp.int32)])
    dst = jnp.concatenate([edge_index[1], jnp.zeros((pad,), jnp.int32)])
    w = jnp.concatenate([edge_attr, jnp.zeros((pad,), jnp.float32)])
    srcp = src.reshape(TOTCH, CHUNK)
    dstp = dst.reshape(TOTCH, CHUNK)
    wp = w.reshape(TOTCH, CHUNK)
    z1d = jnp.zeros((DPT,), jnp.float32)

    deg2 = _deg_kernel(dstp, wp, z1d)
    degA = deg2[0, :N].reshape(N, 1)
    degB = deg2[1, :N].reshape(N, 1)

    g1, xr1, dis = _prep(x, Wi1, Wr1, b1.reshape(1, D), degA, degB)
    agg1 = _agg_kernel(g1, srcp, dstp, wp)
    g2, xr2 = _mid(agg1[0], agg1[1], xr1, dis, Wi2, Wr2, b2.reshape(1, D))
    agg2 = _agg_kernel(g2, srcp, dstp, wp)
    g3, xr3 = _mid(agg2[0], agg2[1], xr2, dis, Wi3, Wr3, b3.reshape(1, D))
    agg3 = _agg_kernel(g3, srcp, dstp, wp)
    return _fin(agg3[0], agg3[1], xr3, dis)
